# baseline (speedup 1.0000x reference)
import functools

import jax
import jax.numpy as jnp
from jax import lax
from jax.experimental import pallas as pl
from jax.experimental.pallas import tpu as pltpu
from jax.experimental.pallas import tpu_sc as plsc

_N = 10000
_E = 320000
_D_IN = 128
_D_H = 256
_NPAD = 10240

_EPT_DEG = _E // 32
_EPT = _E // 16
_K = 128
_EPTP = 20480
_NCH = _EPTP // _K
_GRP = 16

_BLK = 512
_NBLK = _NPAD // _BLK

_sc_mesh = plsc.VectorSubcoreMesh(core_axis_name="c", subcore_axis_name="s")


def _rsqrt16(x):
    xi = plsc.bitcast(x, jnp.int32)
    yi = jnp.full((16,), 0x5F3759DF, jnp.int32) - lax.shift_right_logical(xi, 1)
    y = plsc.bitcast(yi, jnp.float32)
    half = x * 0.5
    for _ in range(3):
        y = y * (1.5 - half * y * y)
    return jnp.where(x > 0, y, 0.0)


@functools.partial(
    pl.kernel,
    out_type=(
        jax.ShapeDtypeStruct((2, _NPAD), jnp.float32),
        jax.ShapeDtypeStruct((32, _EPT_DEG), jnp.float32),
    ),
    mesh=_sc_mesh,
    compiler_params=pltpu.CompilerParams(needs_layout_passes=False),
    scratch_types=[
        pltpu.VMEM((_EPT,), jnp.int32),
        pltpu.VMEM((_EPT,), jnp.float32),
        pltpu.VMEM((_NPAD,), jnp.float32),
        pltpu.VMEM((16, 640), jnp.float32),
        pltpu.VMEM((640,), jnp.float32),
        pltpu.VMEM((_NPAD,), jnp.float32),
        pltpu.VMEM((_EPT_DEG,), jnp.int32),
        pltpu.VMEM((_EPT_DEG,), jnp.float32),
        pltpu.VMEM((_EPT_DEG,), jnp.float32),
        pltpu.VMEM_SHARED((16, _NPAD), jnp.float32),
        pltpu.VMEM_SHARED((_NPAD,), jnp.float32),
    ],
)
def _prep_pass(dst16_hbm, ew16_hbm, src32_hbm, ew32_hbm, dis_hbm, wc_hbm,
               idx_v, ew_v, acc_v, red_v, stripe_v, dis_v, src10_v, ew10_v,
               wc10_v, part_sh, dis_sh):
    c = lax.axis_index("c")
    s = lax.axis_index("s")
    wid = c * 16 + s
    zeros = jnp.zeros((16,), jnp.float32)

    def zb(i, carry):
        acc_v[pl.ds(i * 16, 16)] = zeros
        return carry

    lax.fori_loop(0, _NPAD // 16, zb, 0)
    pltpu.sync_copy(dst16_hbm.at[s], idx_v)
    pltpu.sync_copy(ew16_hbm.at[s], ew_v)

    def eb(i, carry):
        plsc.addupdate_scatter(acc_v, [idx_v[pl.ds(i * 16, 16)]],
                               ew_v[pl.ds(i * 16, 16)])
        return carry

    lax.fori_loop(0, _EPT // 16, eb, 0)

    pltpu.sync_copy(acc_v, part_sh.at[s])
    plsc.subcore_barrier()
    pltpu.sync_copy(part_sh.at[:, pl.ds(640 * s, 640)], red_v)

    def rb(k, carry):
        sl = pl.ds(k * 16, 16)
        tot = red_v[0, sl]
        for t in range(1, 16):
            tot = tot + red_v[t, sl]
        stripe_v[sl] = _rsqrt16(tot)
        return carry

    lax.fori_loop(0, 40, rb, 0)
    pltpu.sync_copy(stripe_v, dis_sh.at[pl.ds(640 * s, 640)])
    pltpu.sync_copy(stripe_v, dis_hbm.at[c].at[pl.ds(640 * s, 640)])
    plsc.subcore_barrier()

    pltpu.sync_copy(dis_sh, dis_v)
    pltpu.sync_copy(src32_hbm.at[wid], src10_v)
    pltpu.sync_copy(ew32_hbm.at[wid], ew10_v)

    def wb(i, carry):
        sl = pl.ds(i * 16, 16)
        dsrc = plsc.load_gather(dis_v, [src10_v[sl]])
        wc10_v[sl] = ew10_v[sl] * dsrc
        return carry

    lax.fori_loop(0, _EPT_DEG // 16, wb, 0)
    pltpu.sync_copy(wc10_v, wc_hbm.at[wid])


@functools.partial(
    pl.kernel,
    out_type=jax.ShapeDtypeStruct((2, _NPAD, 128), jnp.float32),
    mesh=_sc_mesh,
    compiler_params=pltpu.CompilerParams(needs_layout_passes=False),
    scratch_types=[
        pltpu.VMEM((_GRP, _K), jnp.int32),
        pltpu.VMEM((_GRP, _K), jnp.int32),
        pltpu.VMEM((_GRP * _K,), jnp.float32),
        pltpu.VMEM((2, _K, 128), jnp.float32),
        pltpu.VMEM_SHARED((_NPAD, 128), jnp.float32),
        pltpu.SemaphoreType.DMA,
        pltpu.SemaphoreType.DMA,
        pltpu.SemaphoreType.DMA,
        pltpu.SemaphoreType.DMA,
    ],
)
def _edge_pass(src_hbm, dst_hbm, ew_hbm, xw_hbm, agg_hbm,
               src_v, dst_v, ew_v, rows_v, acc_sh,
               gsem0, gsem1, ssem0, ssem1):
    c = lax.axis_index("c")
    s = lax.axis_index("s")
    gsems = (gsem0, gsem1)
    ssems = (ssem0, ssem1)

    zeros = jnp.zeros((16,), jnp.float32)

    def zrow(i, carry):
        for j in range(8):
            rows_v[0, i, pl.ds(j * 16, 16)] = zeros
        return carry

    lax.fori_loop(0, _K, zrow, 0)
    for t in range(5):
        pltpu.sync_copy(rows_v.at[0], acc_sh.at[pl.ds(s * 640 + t * _K, _K)])
    plsc.subcore_barrier()

    def scale(m, j):
        @plsc.parallel_loop(0, _K, step=1, unroll=4)
        def _srow(i):
            nb = plsc.load_gather(ew_v, [jnp.full((16,), j * _K + i, jnp.int32)])
            for jj in range(8):
                sl = pl.ds(jj * 16, 16)
                rows_v[m, i, sl] = rows_v[m, i, sl] * nb

    def group_body(g, carry):
        pltpu.sync_copy(src_hbm.at[s].at[g], src_v)
        pltpu.sync_copy(dst_hbm.at[s].at[g], dst_v)
        pltpu.sync_copy(ew_hbm.at[s].at[g], ew_v)
        gd = [None, None]
        sd = [None, None]
        gd[0] = pltpu.async_copy(xw_hbm.at[c].at[src_v.at[0]], rows_v.at[0], gsems[0])
        for j in range(_GRP):
            m = j % 2
            gd[m].wait()
            if j + 1 < _GRP:
                m2 = 1 - m
                if sd[m2] is not None:
                    sd[m2].wait()
                gd[m2] = pltpu.async_copy(xw_hbm.at[c].at[src_v.at[j + 1]],
                                          rows_v.at[m2], gsems[m2])
            scale(m, j)
            sd[m] = pltpu.async_copy(rows_v.at[m], acc_sh.at[dst_v.at[j]],
                                     ssems[m], add=True)
        sd[0].wait()
        sd[1].wait()
        return carry

    lax.fori_loop(0, _NCH // _GRP, group_body, 0)
    plsc.subcore_barrier()
    pltpu.sync_copy(acc_sh.at[pl.ds(s * 640, 640)],
                    agg_hbm.at[c].at[pl.ds(s * 640, 640)])


def _tcA_body(h_ref, w_ref, r_ref, b_ref, xw_ref, hr_ref):
    rows = h_ref[...]
    xw_ref[0] = jnp.dot(rows, w_ref[...], preferred_element_type=jnp.float32)
    hr_ref[...] = (
        jnp.dot(rows, r_ref[...], preferred_element_type=jnp.float32) + b_ref[...]
    )


_tcA = pl.pallas_call(
    _tcA_body,
    grid=(_NBLK, 2),
    in_specs=[
        pl.BlockSpec((_BLK, _D_IN), lambda i, j: (i, 0)),
        pl.BlockSpec((_D_IN, 128), lambda i, j: (0, j)),
        pl.BlockSpec((_D_IN, 128), lambda i, j: (0, j)),
        pl.BlockSpec((1, 128), lambda i, j: (0, j)),
    ],
    out_specs=[
        pl.BlockSpec((1, _BLK, 128), lambda i, j: (j, i, 0)),
        pl.BlockSpec((_BLK, 128), lambda i, j: (i, j)),
    ],
    out_shape=[
        jax.ShapeDtypeStruct((2, _NPAD, 128), jnp.float32),
        jax.ShapeDtypeStruct((_NPAD, _D_H), jnp.float32),
    ],
)


def _tcB_body(agg_ref, hr0_ref, dis_ref, w_ref, r_ref, b_ref, xw_ref, hr_ref):
    dis = dis_ref[pl.ds(pl.program_id(0), 1), :][0]
    h1a = jnp.maximum(agg_ref[0] * dis[:, None] + hr0_ref[:, :128], 0.0)
    h1b = jnp.maximum(agg_ref[1] * dis[:, None] + hr0_ref[:, 128:], 0.0)
    w = w_ref[...]
    xw_ref[0] = (
        jnp.dot(h1a, w[:128], preferred_element_type=jnp.float32)
        + jnp.dot(h1b, w[128:], preferred_element_type=jnp.float32)
    )
    r = r_ref[...]
    hr_ref[...] = (
        jnp.dot(h1a, r[:128], preferred_element_type=jnp.float32)
        + jnp.dot(h1b, r[128:], preferred_element_type=jnp.float32)
        + b_ref[...]
    )


_tcB = pl.pallas_call(
    _tcB_body,
    grid=(_NBLK, 2),
    in_specs=[
        pl.BlockSpec((2, _BLK, 128), lambda i, j: (0, i, 0)),
        pl.BlockSpec((_BLK, _D_H), lambda i, j: (i, 0)),
        pl.BlockSpec((_NBLK, _BLK), lambda i, j: (0, 0)),
        pl.BlockSpec((_D_H, 128), lambda i, j: (0, j)),
        pl.BlockSpec((_D_H, 128), lambda i, j: (0, j)),
        pl.BlockSpec((1, 128), lambda i, j: (0, j)),
    ],
    out_specs=[
        pl.BlockSpec((1, _BLK, 128), lambda i, j: (j, i, 0)),
        pl.BlockSpec((_BLK, 128), lambda i, j: (i, j)),
    ],
    out_shape=[
        jax.ShapeDtypeStruct((2, _NPAD, 128), jnp.float32),
        jax.ShapeDtypeStruct((_NPAD, _D_H), jnp.float32),
    ],
)


def _tcC_body(agg_ref, hr1_ref, dis_ref, h2_ref):
    dis = dis_ref[pl.ds(pl.program_id(0), 1), :][0]
    h2_ref[...] = jnp.maximum(agg_ref[0] * dis[:, None] + hr1_ref[...], 0.0)


_tcC = pl.pallas_call(
    _tcC_body,
    grid=(_NBLK, 2),
    in_specs=[
        pl.BlockSpec((1, _BLK, 128), lambda i, j: (j, i, 0)),
        pl.BlockSpec((_BLK, 128), lambda i, j: (i, j)),
        pl.BlockSpec((_NBLK, _BLK), lambda i, j: (0, 0)),
    ],
    out_specs=pl.BlockSpec((_BLK, 128), lambda i, j: (i, j)),
    out_shape=jax.ShapeDtypeStruct((_NPAD, _D_H), jnp.float32),
)


def kernel(h, edge_index, edge_weight, W0, R0, b0, W1, R1, b1):
    src = edge_index[0]
    dst = edge_index[1]
    dst16 = dst.reshape(16, _EPT)
    ew16f = edge_weight.reshape(16, _EPT)
    src32 = src.reshape(32, _EPT_DEG)
    ew32 = edge_weight.reshape(32, _EPT_DEG)
    pad16 = ((0, 0), (0, _EPTP - _EPT))
    src16 = jnp.pad(src.reshape(16, _EPT), pad16).reshape(
        16, _NCH // _GRP, _GRP, _K)
    dst16c = jnp.pad(dst.reshape(16, _EPT), pad16).reshape(
        16, _NCH // _GRP, _GRP, _K)
    b0r = b0.reshape(1, _D_H)
    b1r = b1.reshape(1, _D_H)
    h_pad = jnp.pad(h, ((0, _NPAD - _N), (0, 0)))

    dis2, wcomb = _prep_pass(dst16, ew16f, src32, ew32)
    dis_b = dis2[0].reshape(_NBLK, _BLK)
    wc16 = jnp.pad(wcomb.reshape(16, _EPT), pad16).reshape(
        16, _NCH // _GRP, _GRP * _K)
    xw0, hr0 = _tcA(h_pad, W0, R0, b0r)
    agg0 = _edge_pass(src16, dst16c, wc16, xw0)
    xw1, hr1 = _tcB(agg0, hr0, dis_b, W1, R1, b1r)
    agg1 = _edge_pass(src16, dst16c, wc16, xw1)
    return _tcC(agg1, hr1, dis_b)[:_N]


# speedup vs baseline: 1.9046x; 1.9046x over previous
import functools

import jax
import jax.numpy as jnp
from jax import lax
from jax.experimental import pallas as pl
from jax.experimental.pallas import tpu as pltpu
from jax.experimental.pallas import tpu_sc as plsc

_N = 10000
_E = 320000
_D_IN = 128
_D_H = 256
_NPAD = 10240

_EPT_DEG = _E // 32
_EPT = _E // 16
_K = 80
_NCH = _EPT // _K
_GRP = 50

_BLK = 512
_NBLK = _NPAD // _BLK

_sc_mesh = plsc.VectorSubcoreMesh(core_axis_name="c", subcore_axis_name="s")


def _rsqrt16(x):
    xi = plsc.bitcast(x, jnp.int32)
    yi = jnp.full((16,), 0x5F3759DF, jnp.int32) - lax.shift_right_logical(xi, 1)
    y = plsc.bitcast(yi, jnp.float32)
    half = x * 0.5
    for _ in range(3):
        y = y * (1.5 - half * y * y)
    return jnp.where(x > 0, y, 0.0)


@functools.partial(
    pl.kernel,
    out_type=(
        jax.ShapeDtypeStruct((2, _NPAD), jnp.float32),
        jax.ShapeDtypeStruct((32, _EPT_DEG), jnp.float32),
    ),
    mesh=_sc_mesh,
    compiler_params=pltpu.CompilerParams(needs_layout_passes=False),
    scratch_types=[
        pltpu.VMEM((_EPT,), jnp.int32),
        pltpu.VMEM((_EPT,), jnp.float32),
        pltpu.VMEM((_NPAD,), jnp.float32),
        pltpu.VMEM((16, 640), jnp.float32),
        pltpu.VMEM((640,), jnp.float32),
        pltpu.VMEM((_NPAD,), jnp.float32),
        pltpu.VMEM((_EPT_DEG,), jnp.int32),
        pltpu.VMEM((_EPT_DEG,), jnp.float32),
        pltpu.VMEM((_EPT_DEG,), jnp.float32),
        pltpu.VMEM_SHARED((16, _NPAD), jnp.float32),
        pltpu.VMEM_SHARED((_NPAD,), jnp.float32),
    ],
)
def _prep_pass(dst16_hbm, ew16_hbm, src32_hbm, ew32_hbm, dis_hbm, wc_hbm,
               idx_v, ew_v, acc_v, red_v, stripe_v, dis_v, src10_v, ew10_v,
               wc10_v, part_sh, dis_sh):
    c = lax.axis_index("c")
    s = lax.axis_index("s")
    wid = c * 16 + s
    zeros = jnp.zeros((16,), jnp.float32)

    def zb(i, carry):
        acc_v[pl.ds(i * 16, 16)] = zeros
        return carry

    lax.fori_loop(0, _NPAD // 16, zb, 0)
    pltpu.sync_copy(dst16_hbm.at[s], idx_v)
    pltpu.sync_copy(ew16_hbm.at[s], ew_v)

    def eb(i, carry):
        plsc.addupdate_scatter(acc_v, [idx_v[pl.ds(i * 16, 16)]],
                               ew_v[pl.ds(i * 16, 16)])
        return carry

    lax.fori_loop(0, _EPT // 16, eb, 0)

    pltpu.sync_copy(acc_v, part_sh.at[s])
    plsc.subcore_barrier()
    pltpu.sync_copy(part_sh.at[:, pl.ds(640 * s, 640)], red_v)

    def rb(k, carry):
        sl = pl.ds(k * 16, 16)
        tot = red_v[0, sl]
        for t in range(1, 16):
            tot = tot + red_v[t, sl]
        stripe_v[sl] = _rsqrt16(tot)
        return carry

    lax.fori_loop(0, 40, rb, 0)
    pltpu.sync_copy(stripe_v, dis_sh.at[pl.ds(640 * s, 640)])
    pltpu.sync_copy(stripe_v, dis_hbm.at[c].at[pl.ds(640 * s, 640)])
    plsc.subcore_barrier()

    pltpu.sync_copy(dis_sh, dis_v)
    pltpu.sync_copy(src32_hbm.at[wid], src10_v)
    pltpu.sync_copy(ew32_hbm.at[wid], ew10_v)

    def wb(i, carry):
        sl = pl.ds(i * 16, 16)
        dsrc = plsc.load_gather(dis_v, [src10_v[sl]])
        wc10_v[sl] = ew10_v[sl] * dsrc
        return carry

    lax.fori_loop(0, _EPT_DEG // 16, wb, 0)
    pltpu.sync_copy(wc10_v, wc_hbm.at[wid])


@functools.partial(
    pl.kernel,
    out_type=jax.ShapeDtypeStruct((2, _NPAD, 128), jnp.float32),
    mesh=_sc_mesh,
    compiler_params=pltpu.CompilerParams(needs_layout_passes=False),
    scratch_types=[
        pltpu.VMEM((_GRP, _K), jnp.int32),
        pltpu.VMEM((_GRP, _K), jnp.int32),
        pltpu.VMEM((_GRP * _K,), jnp.float32),
        pltpu.VMEM((3, _K, 128), jnp.float32),
        pltpu.VMEM_SHARED((_NPAD, 128), jnp.float32),
        pltpu.SemaphoreType.DMA,
        pltpu.SemaphoreType.DMA,
        pltpu.SemaphoreType.DMA,
        pltpu.SemaphoreType.DMA,
        pltpu.SemaphoreType.DMA,
        pltpu.SemaphoreType.DMA,
    ],
)
def _edge_pass(src_hbm, dst_hbm, ew_hbm, xw_hbm, agg_hbm,
               src_v, dst_v, ew_v, rows_v, acc_sh,
               gsem0, gsem1, gsem2, ssem0, ssem1, ssem2):
    c = lax.axis_index("c")
    s = lax.axis_index("s")
    gsems = (gsem0, gsem1, gsem2)
    ssems = (ssem0, ssem1, ssem2)

    zeros = jnp.zeros((16,), jnp.float32)

    def zrow(i, carry):
        for j in range(8):
            rows_v[0, i, pl.ds(j * 16, 16)] = zeros
        return carry

    lax.fori_loop(0, _K, zrow, 0)
    for t in range(8):
        pltpu.sync_copy(rows_v.at[0], acc_sh.at[pl.ds(s * 640 + t * _K, _K)])
    plsc.subcore_barrier()

    def scale(m, j):
        @plsc.parallel_loop(0, _K, step=1, unroll=8)
        def _srow(i):
            nb = plsc.load_gather(ew_v, [jnp.full((16,), j * _K + i, jnp.int32)])
            for jj in range(8):
                sl = pl.ds(jj * 16, 16)
                rows_v[m, i, sl] = rows_v[m, i, sl] * nb

    def group_body(g, carry):
        pltpu.sync_copy(src_hbm.at[s].at[g], src_v)
        pltpu.sync_copy(dst_hbm.at[s].at[g], dst_v)
        pltpu.sync_copy(ew_hbm.at[s].at[g], ew_v)
        gd = [None, None, None]
        sd = [None, None, None]
        gd[0] = pltpu.async_copy(xw_hbm.at[c].at[src_v.at[0]], rows_v.at[0], gsems[0])
        gd[1] = pltpu.async_copy(xw_hbm.at[c].at[src_v.at[1]], rows_v.at[1], gsems[1])
        for j in range(_GRP):
            m = j % 3
            gd[m].wait()
            scale(m, j)
            sd[m] = pltpu.async_copy(rows_v.at[m], acc_sh.at[dst_v.at[j]],
                                     ssems[m], add=True)
            if j + 2 < _GRP:
                m2 = (j + 2) % 3
                if sd[m2] is not None:
                    sd[m2].wait()
                gd[m2] = pltpu.async_copy(xw_hbm.at[c].at[src_v.at[j + 2]],
                                          rows_v.at[m2], gsems[m2])
        for m in range(3):
            sd[m].wait()
        return carry

    lax.fori_loop(0, _NCH // _GRP, group_body, 0)
    plsc.subcore_barrier()
    pltpu.sync_copy(acc_sh.at[pl.ds(s * 640, 640)],
                    agg_hbm.at[c].at[pl.ds(s * 640, 640)])


def _tcA_body(h_ref, w_ref, r_ref, b_ref, xw_ref, hr_ref):
    rows = h_ref[...]
    xw_ref[0] = jnp.dot(rows, w_ref[...], preferred_element_type=jnp.float32)
    hr_ref[...] = (
        jnp.dot(rows, r_ref[...], preferred_element_type=jnp.float32) + b_ref[...]
    )


_tcA = pl.pallas_call(
    _tcA_body,
    grid=(_NBLK, 2),
    in_specs=[
        pl.BlockSpec((_BLK, _D_IN), lambda i, j: (i, 0)),
        pl.BlockSpec((_D_IN, 128), lambda i, j: (0, j)),
        pl.BlockSpec((_D_IN, 128), lambda i, j: (0, j)),
        pl.BlockSpec((1, 128), lambda i, j: (0, j)),
    ],
    out_specs=[
        pl.BlockSpec((1, _BLK, 128), lambda i, j: (j, i, 0)),
        pl.BlockSpec((_BLK, 128), lambda i, j: (i, j)),
    ],
    out_shape=[
        jax.ShapeDtypeStruct((2, _NPAD, 128), jnp.float32),
        jax.ShapeDtypeStruct((_NPAD, _D_H), jnp.float32),
    ],
)


def _tcB_body(agg_ref, hr0_ref, dis_ref, w_ref, r_ref, b_ref, xw_ref, hr_ref):
    dis = dis_ref[pl.ds(pl.program_id(0), 1), :][0]
    h1a = jnp.maximum(agg_ref[0] * dis[:, None] + hr0_ref[:, :128], 0.0)
    h1b = jnp.maximum(agg_ref[1] * dis[:, None] + hr0_ref[:, 128:], 0.0)
    w = w_ref[...]
    xw_ref[0] = (
        jnp.dot(h1a, w[:128], preferred_element_type=jnp.float32)
        + jnp.dot(h1b, w[128:], preferred_element_type=jnp.float32)
    )
    r = r_ref[...]
    hr_ref[...] = (
        jnp.dot(h1a, r[:128], preferred_element_type=jnp.float32)
        + jnp.dot(h1b, r[128:], preferred_element_type=jnp.float32)
        + b_ref[...]
    )


_tcB = pl.pallas_call(
    _tcB_body,
    grid=(_NBLK, 2),
    in_specs=[
        pl.BlockSpec((2, _BLK, 128), lambda i, j: (0, i, 0)),
        pl.BlockSpec((_BLK, _D_H), lambda i, j: (i, 0)),
        pl.BlockSpec((_NBLK, _BLK), lambda i, j: (0, 0)),
        pl.BlockSpec((_D_H, 128), lambda i, j: (0, j)),
        pl.BlockSpec((_D_H, 128), lambda i, j: (0, j)),
        pl.BlockSpec((1, 128), lambda i, j: (0, j)),
    ],
    out_specs=[
        pl.BlockSpec((1, _BLK, 128), lambda i, j: (j, i, 0)),
        pl.BlockSpec((_BLK, 128), lambda i, j: (i, j)),
    ],
    out_shape=[
        jax.ShapeDtypeStruct((2, _NPAD, 128), jnp.float32),
        jax.ShapeDtypeStruct((_NPAD, _D_H), jnp.float32),
    ],
)


def _tcC_body(agg_ref, hr1_ref, dis_ref, h2_ref):
    dis = dis_ref[pl.ds(pl.program_id(0), 1), :][0]
    h2_ref[...] = jnp.maximum(agg_ref[0] * dis[:, None] + hr1_ref[...], 0.0)


_tcC = pl.pallas_call(
    _tcC_body,
    grid=(_NBLK, 2),
    in_specs=[
        pl.BlockSpec((1, _BLK, 128), lambda i, j: (j, i, 0)),
        pl.BlockSpec((_BLK, 128), lambda i, j: (i, j)),
        pl.BlockSpec((_NBLK, _BLK), lambda i, j: (0, 0)),
    ],
    out_specs=pl.BlockSpec((_BLK, 128), lambda i, j: (i, j)),
    out_shape=jax.ShapeDtypeStruct((_NPAD, _D_H), jnp.float32),
)


def kernel(h, edge_index, edge_weight, W0, R0, b0, W1, R1, b1):
    src = edge_index[0]
    dst = edge_index[1]
    dst16 = dst.reshape(16, _EPT)
    ew16f = edge_weight.reshape(16, _EPT)
    src32 = src.reshape(32, _EPT_DEG)
    ew32 = edge_weight.reshape(32, _EPT_DEG)
    src16 = src.reshape(16, _NCH // _GRP, _GRP, _K)
    dst16c = dst.reshape(16, _NCH // _GRP, _GRP, _K)
    b0r = b0.reshape(1, _D_H)
    b1r = b1.reshape(1, _D_H)
    h_pad = jnp.pad(h, ((0, _NPAD - _N), (0, 0)))

    dis2, wcomb = _prep_pass(dst16, ew16f, src32, ew32)
    dis_b = dis2[0].reshape(_NBLK, _BLK)
    wc16 = wcomb.reshape(16, _NCH // _GRP, _GRP * _K)
    xw0, hr0 = _tcA(h_pad, W0, R0, b0r)
    agg0 = _edge_pass(src16, dst16c, wc16, xw0)
    xw1, hr1 = _tcB(agg0, hr0, dis_b, W1, R1, b1r)
    agg1 = _edge_pass(src16, dst16c, wc16, xw1)
    return _tcC(agg1, hr1, dis_b)[:_N]
